# baseline (speedup 1.0000x reference)
import functools

import jax
import jax.numpy as jnp
from jax import lax
from jax.experimental import pallas as pl
from jax.experimental.pallas import tpu as pltpu
from jax.experimental.pallas import tpu_sc as plsc

D_MODEL = 64
SCALE = float(D_MODEL) ** 0.5

NUM_WORKERS = 32
NBUF = 4


def _emb_kernel(rows_per_w, seq, idx_hbm, table_hbm, out_hbm, idx_v,
                gbufs, wbufs, gsems, wsems):
    nc = 2
    wid = lax.axis_index("s") * nc + lax.axis_index("c")
    row0 = wid * rows_per_w
    pltpu.sync_copy(idx_hbm.at[pl.ds(row0, rows_per_w)], idx_v)

    def gather(j, b):
        return pltpu.make_async_copy(
            table_hbm.at[idx_v.at[j]], gbufs[b], gsems[b]
        )

    def write(j, b):
        return pltpu.make_async_copy(wbufs[b], out_hbm.at[row0 + j], wsems[b])

    for b in range(NBUF):
        gather(b, b).start()

    n_rounds = rows_per_w // NBUF

    def round_body(k, _):
        for b in range(NBUF):
            j = k * NBUF + b
            gather(j, b).wait()
            @pl.when(k > 0)
            def _():
                write(j - NBUF, b).wait()

            @plsc.parallel_loop(0, seq, unroll=4)
            def _(i):
                for t in range(D_MODEL // 16):
                    sl = pl.ds(t * 16, 16)
                    wbufs[b][i, sl] = gbufs[b][i, sl] * SCALE

            @pl.when(j + NBUF < rows_per_w)
            def _():
                gather(j + NBUF, b).start()

            write(j, b).start()
        return 0

    lax.fori_loop(0, n_rounds, round_body, 0)

    for b in range(NBUF):
        write(rows_per_w - NBUF + b, b).wait()


def kernel(x, table):
    b0, seq = x.shape
    rows_per_w = b0 // NUM_WORKERS
    assert rows_per_w * NUM_WORKERS == b0 and rows_per_w % NBUF == 0
    assert seq <= 128

    mesh = plsc.VectorSubcoreMesh(core_axis_name="c", subcore_axis_name="s")
    out = pl.kernel(
        functools.partial(_emb_kernel, rows_per_w, seq),
        mesh=mesh,
        compiler_params=pltpu.CompilerParams(use_tc_tiling_on_sc=False),
        out_type=jax.ShapeDtypeStruct((b0, seq, D_MODEL), jnp.float32),
        scratch_types=[
            pltpu.VMEM((rows_per_w, seq), jnp.int32),
            [pltpu.VMEM((seq, D_MODEL), jnp.float32) for _ in range(NBUF)],
            [pltpu.VMEM((seq, D_MODEL), jnp.float32) for _ in range(NBUF)],
            [pltpu.SemaphoreType.DMA for _ in range(NBUF)],
            [pltpu.SemaphoreType.DMA for _ in range(NBUF)],
        ],
    )(x.astype(jnp.int32), table)
    return out


# speedup vs baseline: 1.2243x; 1.2243x over previous
import functools

import jax
import jax.numpy as jnp
from jax import lax
from jax.experimental import pallas as pl
from jax.experimental.pallas import tpu as pltpu
from jax.experimental.pallas import tpu_sc as plsc

D_MODEL = 64
SCALE = float(D_MODEL) ** 0.5
NUM_WORKERS = 32
STEP = 128
LANES = 16


def _emb_kernel(steps_per_w, idx_hbm, table_hbm, out_hbm,
                idx_v, gbuf, wbuf, gsem, wsem):
    nc = 2
    wid = lax.axis_index("s") * nc + lax.axis_index("c")
    per_w = steps_per_w * STEP
    base = wid * per_w
    pltpu.sync_copy(idx_hbm.at[pl.ds(base, per_w)], idx_v)

    def step_body(j, _):
        def row16(c, _):
            r0 = c * LANES
            chunk = idx_v[pl.ds(j * STEP + r0, LANES)]
            for l in range(LANES):
                pltpu.make_async_copy(
                    table_hbm.at[chunk[l]], gbuf.at[r0 + l], gsem
                ).start()
            return 0

        lax.fori_loop(0, STEP // LANES, row16, 0)
        pltpu.make_async_copy(
            out_hbm.at[pl.ds(0, STEP)], gbuf, gsem
        ).wait()

        def srow(g, _):
            for t in range(D_MODEL // 16):
                sl = pl.ds(t * 16, 16)
                wbuf[g, sl] = gbuf[g, sl] * SCALE
            return 0

        lax.fori_loop(0, STEP, srow, 0)
        out_slab = out_hbm.at[pl.ds(base + j * STEP, STEP)]
        pltpu.make_async_copy(wbuf, out_slab, wsem).start()
        pltpu.make_async_copy(wbuf, out_slab, wsem).wait()
        return 0

    lax.fori_loop(0, steps_per_w, step_body, 0)


def kernel(x, table):
    b0, b1 = x.shape
    total = b0 * b1
    n_steps = total // STEP
    steps_per_w = n_steps // NUM_WORKERS
    assert n_steps * STEP == total and steps_per_w * NUM_WORKERS == n_steps
    idx1d = x.reshape(total).astype(jnp.int32)

    mesh = plsc.VectorSubcoreMesh(core_axis_name="c", subcore_axis_name="s")
    out = pl.kernel(
        functools.partial(_emb_kernel, steps_per_w),
        mesh=mesh,
        out_type=jax.ShapeDtypeStruct((total, D_MODEL), jnp.float32),
        scratch_types=[
            pltpu.VMEM((6400,), jnp.int32),
            pltpu.VMEM((STEP, D_MODEL), jnp.float32),
            pltpu.VMEM((STEP, D_MODEL), jnp.float32),
            pltpu.SemaphoreType.DMA,
            pltpu.SemaphoreType.DMA,
        ],
    )(idx1d, table)
    return out.reshape(b0, b1, D_MODEL)
